# Initial kernel scaffold; baseline (speedup 1.0000x reference)
import functools

import jax
import jax.numpy as jnp
from jax import lax
from jax.experimental import pallas as pl
from jax.experimental.pallas import tpu as pltpu
from jax.experimental.pallas import tpu_sc as plsc

B = 1024
AD = 512
NF = 64
NLB = 128
NACT = 256
HID = 1024
IN1 = NLB + AD * NF + AD

NW = 32
SPW = B // NW

R = B * AD
RT = 4096
NRT = R // RT

NKP = 256
NK = 66


def _sc_counts(active_as):
    mesh = plsc.VectorSubcoreMesh(core_axis_name="c", subcore_axis_name="s")

    @functools.partial(
        pl.kernel,
        out_type=jax.ShapeDtypeStruct((B, AD), jnp.float32),
        mesh=mesh,
        scratch_types=[
            pltpu.VMEM((SPW, NACT), jnp.int32),
            pltpu.VMEM((SPW, AD), jnp.float32),
        ],
    )
    def k(act_hbm, out_hbm, act_v, cnt_v):
        wid = lax.axis_index("s") * 2 + lax.axis_index("c")
        base = wid * SPW
        pltpu.sync_copy(act_hbm.at[pl.ds(base, SPW)], act_v)
        zeros16 = jnp.zeros((16,), jnp.float32)
        ones16 = jnp.ones((16,), jnp.float32)

        def zero_body(s, _):
            for v in range(AD // 16):
                cnt_v[s, pl.ds(v * 16, 16)] = zeros16
            return 0

        lax.fori_loop(0, SPW, zero_body, 0)

        def scat_body(s, _):
            srow = jnp.full((16,), s, jnp.int32)
            for v in range(NACT // 16):
                idx = act_v[s, pl.ds(v * 16, 16)]
                plsc.addupdate_scatter(cnt_v, [srow, idx], ones16)
            return 0

        lax.fori_loop(0, SPW, scat_body, 0)
        pltpu.sync_copy(cnt_v, out_hbm.at[pl.ds(base, SPW)])

    return k(active_as)


def _stats_body(c_ref, f_ref, flb_ref, m_ref, st_ref, acc):
    r = pl.program_id(0)

    @pl.when(r == 0)
    def _():
        acc[...] = jnp.zeros((8, NF), jnp.float32)

    cb = c_ref[0]
    fb = f_ref[...]
    m_ref[0] = jnp.minimum(cb, 1.0)
    dn = (((1,), (0,)), ((), ()))
    s1 = lax.dot_general(cb, fb, dn, precision=lax.Precision.HIGHEST,
                         preferred_element_type=jnp.float32)
    s2 = lax.dot_general(cb, fb * fb, dn, precision=lax.Precision.HIGHEST,
                         preferred_element_type=jnp.float32)
    acc[0:1, :] += s1
    acc[1:2, :] += s2

    @pl.when(r == NRT - 1)
    def _():
        flb = flb_ref[...]
        head = flb[:, :NF]
        tail = flb[:, NF:]
        acc[2:3, :] += jnp.sum(head, axis=0, keepdims=True)
        acc[3:4, :] += jnp.sum(head * head, axis=0, keepdims=True)
        acc[4:5, :] += jnp.sum(tail, axis=0, keepdims=True)
        acc[5:6, :] += jnp.sum(tail * tail, axis=0, keepdims=True)
        st_ref[...] = acc[...]


def _stats_call(c3, f_r, flb):
    return pl.pallas_call(
        _stats_body,
        grid=(NRT,),
        in_specs=[
            pl.BlockSpec((1, 1, RT), lambda r: (r, 0, 0)),
            pl.BlockSpec((RT, NF), lambda r: (r, 0)),
            pl.BlockSpec((B, NLB), lambda r: (0, 0)),
        ],
        out_specs=[
            pl.BlockSpec((1, 1, RT), lambda r: (r, 0, 0)),
            pl.BlockSpec((8, NF), lambda r: (0, 0)),
        ],
        out_shape=[
            jax.ShapeDtypeStruct((NRT, 1, RT), jnp.float32),
            jax.ShapeDtypeStruct((8, NF), jnp.float32),
        ],
        scratch_shapes=[pltpu.VMEM((8, NF), jnp.float32)],
    )(c3, f_r, flb)


def _mm_body(f0, f1, f2, f3, m0, m1, m2, m3, xs, w1, at, bt, b1r, lnw, lnb,
             w4, b4r, out_ref, acc):
    k = pl.program_id(0)

    @pl.when(k == 0)
    def _():
        acc[...] = jnp.zeros((B, HID), jnp.float32)

    def piece(f_ref, m_ref):
        m2d = m_ref[0]
        mex = jnp.concatenate(
            [jnp.broadcast_to(m2d[:, 0:1], (B, NF)),
             jnp.broadcast_to(m2d[:, 1:2], (B, NF))], axis=1)
        return mex * (f_ref[...] * at[...] + bt[...])

    dn = (((1,), (0,)), ((), ()))

    def accum(x, w):
        acc[...] += lax.dot_general(x, w, dn,
                                    preferred_element_type=jnp.float32)

    @pl.when(k == 0)
    def _():
        x = jnp.concatenate([xs[:, 0:128], piece(f1, m1), piece(f2, m2),
                             piece(f3, m3)], axis=1)
        accum(x, w1[...])

    @pl.when((k >= 1) & (k <= 63))
    def _():
        x = jnp.concatenate([piece(f0, m0), piece(f1, m1), piece(f2, m2),
                             piece(f3, m3)], axis=1)
        accum(x, w1[...])

    @pl.when(k == 64)
    def _():
        x = jnp.concatenate([piece(f0, m0), xs[:, 128:512]], axis=1)
        accum(x, w1[...])

    @pl.when(k == NK - 1)
    def _():
        accum(xs[:, 512:640], w1[0:128, :])
        h = acc[...] + b1r[...]
        h = jnp.where(h > 0, h, jnp.exp(jnp.minimum(h, 0.0)) - 1.0)
        mu = jnp.mean(h, axis=1, keepdims=True)
        hc = h - mu
        var = jnp.mean(hc * hc, axis=1, keepdims=True)
        hn = hc * lax.rsqrt(var + 1e-5) * lnw[...] + lnb[...]
        out_ref[...] = lax.dot_general(
            hn, w4[...], dn, preferred_element_type=jnp.float32) + b4r[...]


def _mm_call(f2d, m_r, xsmall, W1, at, bt, b1r, lnw, lnb, W4, b4r):
    def fmap(j):
        return lambda k: (0, jnp.clip(4 * k - 1 + j, 0, NKP - 1))

    def mmap(j):
        return lambda k: (jnp.clip(4 * k - 1 + j, 0, NKP - 1), 0, 0)

    def full(shape):
        return pl.BlockSpec(shape, lambda k: tuple(0 for _ in shape))

    return pl.pallas_call(
        _mm_body,
        grid=(NK,),
        in_specs=[
            pl.BlockSpec((B, 128), fmap(0)),
            pl.BlockSpec((B, 128), fmap(1)),
            pl.BlockSpec((B, 128), fmap(2)),
            pl.BlockSpec((B, 128), fmap(3)),
            pl.BlockSpec((1, B, 2), mmap(0)),
            pl.BlockSpec((1, B, 2), mmap(1)),
            pl.BlockSpec((1, B, 2), mmap(2)),
            pl.BlockSpec((1, B, 2), mmap(3)),
            full((B, 640)),
            pl.BlockSpec((512, HID), lambda k: (k, 0)),
            full((1, 128)),
            full((1, 128)),
            full((1, HID)),
            full((1, HID)),
            full((1, HID)),
            full((HID, AD)),
            full((1, AD)),
        ],
        out_specs=pl.BlockSpec((B, AD), lambda k: (0, 0)),
        out_shape=jax.ShapeDtypeStruct((B, AD), jnp.float32),
        scratch_shapes=[pltpu.VMEM((B, HID), jnp.float32)],
    )(f2d, f2d, f2d, f2d, m_r, m_r, m_r, m_r, xsmall, W1, at, bt, b1r,
      lnw, lnb, W4, b4r)


def kernel(feature_lb, feature_as, action, active_as, bn_as_w, bn_as_b,
           bn_lb_w, bn_lb_b, W1, b1, ln1_w, ln1_b, W4, b4):
    c = _sc_counts(active_as)

    f_r = feature_as.reshape(R, NF)
    c3 = c.reshape(NRT, 1, RT)
    m3, st = _stats_call(c3, f_r, feature_lb)

    n_as = jnp.float32(B + B * NACT)
    mean_as = (st[0] + st[2]) / n_as
    var_as = (st[1] + st[3]) / n_as - mean_as * mean_as
    alpha_as = bn_as_w * lax.rsqrt(var_as + 1e-5)
    beta_as = bn_as_b - mean_as * alpha_as

    mean_lb = st[4] / B
    var_lb = st[5] / B - mean_lb * mean_lb
    alpha_lb = bn_lb_w * lax.rsqrt(var_lb + 1e-5)
    beta_lb = bn_lb_b - mean_lb * alpha_lb

    xsmall = jnp.concatenate(
        [feature_lb[:, NF:] * alpha_lb + beta_lb,
         feature_lb[:, :NF] * alpha_as + beta_as,
         action], axis=1)

    m_r = m3.reshape(B, AD // 2, 2).transpose(1, 0, 2)
    at = jnp.tile(alpha_as, 2)[None, :]
    bt = jnp.tile(beta_as, 2)[None, :]

    f2d = feature_as.reshape(B, AD * NF)
    return _mm_call(f2d, m_r, xsmall, W1, at, bt, b1[None, :],
                    ln1_w[None, :], ln1_b[None, :], W4, b4[None, :])


# speedup vs baseline: 3.3555x; 3.3555x over previous
import functools

import jax
import jax.numpy as jnp
from jax import lax
from jax.experimental import pallas as pl
from jax.experimental.pallas import tpu as pltpu
from jax.experimental.pallas import tpu_sc as plsc

B = 1024
AD = 512
NF = 64
NLB = 128
NACT = 256
HID = 1024
IN1 = NLB + AD * NF + AD

NW = 32
SPW = B // NW

R = B * AD
RT = 4096
NRT = R // RT

NKP = 256
NK = 66


def _sc_counts(active_as):
    mesh = plsc.VectorSubcoreMesh(core_axis_name="c", subcore_axis_name="s")

    @functools.partial(
        pl.kernel,
        out_type=jax.ShapeDtypeStruct((B * AD,), jnp.float32),
        mesh=mesh,
        compiler_params=pltpu.CompilerParams(needs_layout_passes=False),
        scratch_types=[
            pltpu.VMEM((SPW * NACT,), jnp.int32),
            pltpu.VMEM((SPW * AD,), jnp.float32),
        ],
    )
    def k(act_hbm, out_hbm, act_v, cnt_v):
        wid = lax.axis_index("s") * 2 + lax.axis_index("c")
        pltpu.sync_copy(act_hbm.at[pl.ds(wid * SPW * NACT, SPW * NACT)],
                        act_v)
        zeros16 = jnp.zeros((16,), jnp.float32)
        ones16 = jnp.ones((16,), jnp.float32)

        def zero_body(i, _):
            cnt_v[pl.ds(i * 16, 16)] = zeros16
            return 0

        lax.fori_loop(0, SPW * AD // 16, zero_body, 0)

        def scat_body(s, _):
            srow = jnp.full((16,), s * AD, jnp.int32)
            for v in range(NACT // 16):
                idx = act_v[pl.ds(s * NACT + v * 16, 16)]
                plsc.addupdate_scatter(cnt_v, [srow + idx], ones16)
            return 0

        lax.fori_loop(0, SPW, scat_body, 0)
        pltpu.sync_copy(cnt_v, out_hbm.at[pl.ds(wid * SPW * AD, SPW * AD)])

    return k(active_as.reshape(B * NACT))


def _stats_body(c_ref, f_ref, flb_ref, m_ref, st_ref, acc):
    r = pl.program_id(0)

    @pl.when(r == 0)
    def _():
        acc[...] = jnp.zeros((8, NF), jnp.float32)

    cb = c_ref[0]
    fb = f_ref[...]
    m_ref[0] = jnp.minimum(cb, 1.0)
    dn = (((1,), (0,)), ((), ()))
    s1 = lax.dot_general(cb, fb, dn, precision=lax.Precision.HIGHEST,
                         preferred_element_type=jnp.float32)
    s2 = lax.dot_general(cb, fb * fb, dn, precision=lax.Precision.HIGHEST,
                         preferred_element_type=jnp.float32)
    acc[0:1, :] += s1
    acc[1:2, :] += s2

    @pl.when(r == NRT - 1)
    def _():
        flb = flb_ref[...]
        head = flb[:, :NF]
        tail = flb[:, NF:]
        acc[2:3, :] += jnp.sum(head, axis=0, keepdims=True)
        acc[3:4, :] += jnp.sum(head * head, axis=0, keepdims=True)
        acc[4:5, :] += jnp.sum(tail, axis=0, keepdims=True)
        acc[5:6, :] += jnp.sum(tail * tail, axis=0, keepdims=True)
        st_ref[...] = acc[...]


def _stats_call(c3, f_r, flb):
    return pl.pallas_call(
        _stats_body,
        grid=(NRT,),
        in_specs=[
            pl.BlockSpec((1, 1, RT), lambda r: (r, 0, 0)),
            pl.BlockSpec((RT, NF), lambda r: (r, 0)),
            pl.BlockSpec((B, NLB), lambda r: (0, 0)),
        ],
        out_specs=[
            pl.BlockSpec((1, 1, RT), lambda r: (r, 0, 0)),
            pl.BlockSpec((8, NF), lambda r: (0, 0)),
        ],
        out_shape=[
            jax.ShapeDtypeStruct((NRT, 1, RT), jnp.float32),
            jax.ShapeDtypeStruct((8, NF), jnp.float32),
        ],
        scratch_shapes=[pltpu.VMEM((8, NF), jnp.float32)],
    )(c3, f_r, flb)


def _mm_body(f0, f1, f2, f3, m0, m1, m2, m3, xs, w1, at, bt, b1r, lnw, lnb,
             w4, b4r, out_ref, acc):
    k = pl.program_id(0)

    @pl.when(k == 0)
    def _():
        acc[...] = jnp.zeros((B, HID), jnp.float32)

    def piece(f_ref, m_ref):
        m2d = m_ref[0]
        mex = jnp.concatenate(
            [jnp.broadcast_to(m2d[:, 0:1], (B, NF)),
             jnp.broadcast_to(m2d[:, 1:2], (B, NF))], axis=1)
        return mex * (f_ref[...] * at[...] + bt[...])

    dn = (((1,), (0,)), ((), ()))

    def accum(x, w):
        acc[...] += lax.dot_general(x, w, dn,
                                    preferred_element_type=jnp.float32)

    @pl.when(k == 0)
    def _():
        x = jnp.concatenate([xs[:, 0:128], piece(f1, m1), piece(f2, m2),
                             piece(f3, m3)], axis=1)
        accum(x, w1[...])

    @pl.when((k >= 1) & (k <= 63))
    def _():
        x = jnp.concatenate([piece(f0, m0), piece(f1, m1), piece(f2, m2),
                             piece(f3, m3)], axis=1)
        accum(x, w1[...])

    @pl.when(k == 64)
    def _():
        x = jnp.concatenate([piece(f0, m0), xs[:, 128:512]], axis=1)
        accum(x, w1[...])

    @pl.when(k == NK - 1)
    def _():
        accum(xs[:, 512:640], w1[0:128, :])
        h = acc[...] + b1r[...]
        h = jnp.where(h > 0, h, jnp.exp(jnp.minimum(h, 0.0)) - 1.0)
        mu = jnp.mean(h, axis=1, keepdims=True)
        hc = h - mu
        var = jnp.mean(hc * hc, axis=1, keepdims=True)
        hn = hc * lax.rsqrt(var + 1e-5) * lnw[...] + lnb[...]
        out_ref[...] = lax.dot_general(
            hn, w4[...], dn, preferred_element_type=jnp.float32) + b4r[...]


def _mm_call(f2d, m_r, xsmall, W1, at, bt, b1r, lnw, lnb, W4, b4r):
    def fmap(j):
        return lambda k: (0, jnp.clip(4 * k - 1 + j, 0, NKP - 1))

    def mmap(j):
        return lambda k: (jnp.clip(4 * k - 1 + j, 0, NKP - 1), 0, 0)

    def full(shape):
        return pl.BlockSpec(shape, lambda k: tuple(0 for _ in shape))

    return pl.pallas_call(
        _mm_body,
        grid=(NK,),
        in_specs=[
            pl.BlockSpec((B, 128), fmap(0)),
            pl.BlockSpec((B, 128), fmap(1)),
            pl.BlockSpec((B, 128), fmap(2)),
            pl.BlockSpec((B, 128), fmap(3)),
            pl.BlockSpec((1, B, 2), mmap(0)),
            pl.BlockSpec((1, B, 2), mmap(1)),
            pl.BlockSpec((1, B, 2), mmap(2)),
            pl.BlockSpec((1, B, 2), mmap(3)),
            full((B, 640)),
            pl.BlockSpec((512, HID), lambda k: (k, 0)),
            full((1, 128)),
            full((1, 128)),
            full((1, HID)),
            full((1, HID)),
            full((1, HID)),
            full((HID, AD)),
            full((1, AD)),
        ],
        out_specs=pl.BlockSpec((B, AD), lambda k: (0, 0)),
        out_shape=jax.ShapeDtypeStruct((B, AD), jnp.float32),
        scratch_shapes=[pltpu.VMEM((B, HID), jnp.float32)],
    )(f2d, f2d, f2d, f2d, m_r, m_r, m_r, m_r, xsmall, W1, at, bt, b1r,
      lnw, lnb, W4, b4r)


def kernel(feature_lb, feature_as, action, active_as, bn_as_w, bn_as_b,
           bn_lb_w, bn_lb_b, W1, b1, ln1_w, ln1_b, W4, b4):
    c = _sc_counts(active_as)

    f_r = feature_as.reshape(R, NF)
    c3 = c.reshape(NRT, 1, RT)
    m3, st = _stats_call(c3, f_r, feature_lb)

    n_as = jnp.float32(B + B * NACT)
    mean_as = (st[0] + st[2]) / n_as
    var_as = (st[1] + st[3]) / n_as - mean_as * mean_as
    alpha_as = bn_as_w * lax.rsqrt(var_as + 1e-5)
    beta_as = bn_as_b - mean_as * alpha_as

    mean_lb = st[4] / B
    var_lb = st[5] / B - mean_lb * mean_lb
    alpha_lb = bn_lb_w * lax.rsqrt(var_lb + 1e-5)
    beta_lb = bn_lb_b - mean_lb * alpha_lb

    xsmall = jnp.concatenate(
        [feature_lb[:, NF:] * alpha_lb + beta_lb,
         feature_lb[:, :NF] * alpha_as + beta_as,
         action], axis=1)

    m_r = m3.reshape(B, AD // 2, 2).transpose(1, 0, 2)
    at = jnp.tile(alpha_as, 2)[None, :]
    bt = jnp.tile(beta_as, 2)[None, :]

    f2d = feature_as.reshape(B, AD * NF)
    return _mm_call(f2d, m_r, xsmall, W1, at, bt, b1[None, :],
                    ln1_w[None, :], ln1_b[None, :], W4, b4[None, :])


# speedup vs baseline: 4.1540x; 1.2380x over previous
import functools

import jax
import jax.numpy as jnp
from jax import lax
from jax.experimental import pallas as pl
from jax.experimental.pallas import tpu as pltpu
from jax.experimental.pallas import tpu_sc as plsc

B = 1024
AD = 512
NF = 64
NLB = 128
NACT = 256
HID = 1024
IN1 = NLB + AD * NF + AD

NW = 32
SPW = B // NW

R = B * AD
RT = 4096
NRT = R // RT

NKP = 256
NK = 66


def _sc_counts(active_as):
    mesh = plsc.VectorSubcoreMesh(core_axis_name="c", subcore_axis_name="s")

    @functools.partial(
        pl.kernel,
        out_type=(jax.ShapeDtypeStruct((B * AD,), jnp.float32),
                  jax.ShapeDtypeStruct((NKP, B, 2), jnp.float32)),
        mesh=mesh,
        compiler_params=pltpu.CompilerParams(needs_layout_passes=False,
                                             use_tc_tiling_on_sc=False),
        scratch_types=[
            pltpu.VMEM((SPW * NACT,), jnp.int32),
            pltpu.VMEM((SPW * AD,), jnp.float32),
            pltpu.VMEM((NKP, SPW, 2), jnp.float32),
        ],
    )
    def k(act_hbm, cnt_hbm, mr_hbm, act_v, cnt_v, m_v):
        wid = lax.axis_index("s") * 2 + lax.axis_index("c")
        base = wid * SPW
        pltpu.sync_copy(act_hbm.at[pl.ds(wid * SPW * NACT, SPW * NACT)],
                        act_v)
        zeros16 = jnp.zeros((16,), jnp.float32)
        ones16 = jnp.ones((16,), jnp.float32)

        def zero_body(i, _):
            cnt_v[pl.ds(i * 16, 16)] = zeros16
            return 0

        lax.fori_loop(0, SPW * AD // 16, zero_body, 0)

        def scat_body(s, _):
            srow = jnp.full((16,), s * AD, jnp.int32)
            for v in range(NACT // 16):
                idx = act_v[pl.ds(s * NACT + v * 16, 16)]
                plsc.addupdate_scatter(cnt_v, [srow + idx], ones16)
            return 0

        lax.fori_loop(0, SPW, scat_body, 0)

        tt = lax.broadcasted_iota(jnp.int32, (16,), 0)

        def mask_body(s, _):
            svec = jnp.full((16,), s, jnp.int32)
            for v in range(AD // 16):
                cnt = cnt_v[pl.ds(s * AD + v * 16, 16)]
                msk = jnp.minimum(cnt, 1.0)
                a = tt + (v * 16)
                plsc.store_scatter(
                    m_v, [lax.shift_right_logical(a, 1), svec,
                          lax.bitwise_and(a, 1)], msk)
            return 0

        lax.fori_loop(0, SPW, mask_body, 0)
        pltpu.sync_copy(cnt_v, cnt_hbm.at[pl.ds(wid * SPW * AD, SPW * AD)])
        pltpu.sync_copy(m_v, mr_hbm.at[:, pl.ds(base, SPW), :])

    return k(active_as.reshape(B * NACT))


def _stats_body(c_ref, f_ref, flb_ref, st_ref, acc):
    r = pl.program_id(0)

    @pl.when(r == 0)
    def _():
        acc[...] = jnp.zeros((8, NF), jnp.float32)

    cb = c_ref[0]
    fb = f_ref[...]
    dn = (((1,), (0,)), ((), ()))
    s1 = lax.dot_general(cb, fb, dn, precision=lax.Precision.HIGHEST,
                         preferred_element_type=jnp.float32)
    s2 = lax.dot_general(cb, fb * fb, dn, precision=lax.Precision.HIGHEST,
                         preferred_element_type=jnp.float32)
    acc[0:1, :] += s1
    acc[1:2, :] += s2

    @pl.when(r == NRT - 1)
    def _():
        flb = flb_ref[...]
        head = flb[:, :NF]
        tail = flb[:, NF:]
        acc[2:3, :] += jnp.sum(head, axis=0, keepdims=True)
        acc[3:4, :] += jnp.sum(head * head, axis=0, keepdims=True)
        acc[4:5, :] += jnp.sum(tail, axis=0, keepdims=True)
        acc[5:6, :] += jnp.sum(tail * tail, axis=0, keepdims=True)
        st_ref[...] = acc[...]


def _stats_call(c3, f_r, flb):
    return pl.pallas_call(
        _stats_body,
        grid=(NRT,),
        in_specs=[
            pl.BlockSpec((1, 1, RT), lambda r: (r, 0, 0)),
            pl.BlockSpec((RT, NF), lambda r: (r, 0)),
            pl.BlockSpec((B, NLB), lambda r: (0, 0)),
        ],
        out_specs=pl.BlockSpec((8, NF), lambda r: (0, 0)),
        out_shape=jax.ShapeDtypeStruct((8, NF), jnp.float32),
        scratch_shapes=[pltpu.VMEM((8, NF), jnp.float32)],
    )(c3, f_r, flb)


def _mm_body(f0, f1, f2, f3, m0, m1, m2, m3, flb, act, alb, blb, w1, at, bt,
             b1r, lnw, lnb, w4, b4r, out_ref, acc):
    k = pl.program_id(0)

    @pl.when(k == 0)
    def _():
        acc[...] = jnp.zeros((B, HID), jnp.float32)

    def piece(f_ref, m_ref):
        m2d = m_ref[0]
        mex = jnp.concatenate(
            [jnp.broadcast_to(m2d[:, 0:1], (B, NF)),
             jnp.broadcast_to(m2d[:, 1:2], (B, NF))], axis=1)
        return mex * (f_ref[...] * at[...] + bt[...])

    dn = (((1,), (0,)), ((), ()))

    def accum(x, w):
        acc[...] += lax.dot_general(x, w, dn,
                                    preferred_element_type=jnp.float32)

    @pl.when(k == 0)
    def _():
        x0 = jnp.concatenate(
            [flb[:, NF:] * alb[...] + blb[...],
             flb[:, :NF] * at[:, :NF] + bt[:, :NF]], axis=1)
        x = jnp.concatenate([x0, piece(f1, m1), piece(f2, m2),
                             piece(f3, m3)], axis=1)
        accum(x, w1[...])

    @pl.when((k >= 1) & (k <= 63))
    def _():
        x = jnp.concatenate([piece(f0, m0), piece(f1, m1), piece(f2, m2),
                             piece(f3, m3)], axis=1)
        accum(x, w1[...])

    @pl.when(k == 64)
    def _():
        x = jnp.concatenate([piece(f0, m0), act[:, 0:384]], axis=1)
        accum(x, w1[...])

    @pl.when(k == NK - 1)
    def _():
        accum(act[:, 384:512], w1[0:128, :])
        h = acc[...] + b1r[...]
        h = jnp.where(h > 0, h, jnp.exp(jnp.minimum(h, 0.0)) - 1.0)
        mu = jnp.mean(h, axis=1, keepdims=True)
        hc = h - mu
        var = jnp.mean(hc * hc, axis=1, keepdims=True)
        hn = hc * lax.rsqrt(var + 1e-5) * lnw[...] + lnb[...]
        out_ref[...] = lax.dot_general(
            hn, w4[...], dn, preferred_element_type=jnp.float32) + b4r[...]


def _mm_call(f2d, m_r, flb, act, alb, blb, W1, at, bt, b1r, lnw, lnb, W4,
             b4r):
    def fmap(j):
        return lambda k: (0, jnp.clip(4 * k - 1 + j, 0, NKP - 1))

    def mmap(j):
        return lambda k: (jnp.clip(4 * k - 1 + j, 0, NKP - 1), 0, 0)

    def full(shape):
        return pl.BlockSpec(shape, lambda k: tuple(0 for _ in shape))

    return pl.pallas_call(
        _mm_body,
        grid=(NK,),
        in_specs=[
            pl.BlockSpec((B, 128), fmap(0)),
            pl.BlockSpec((B, 128), fmap(1)),
            pl.BlockSpec((B, 128), fmap(2)),
            pl.BlockSpec((B, 128), fmap(3)),
            pl.BlockSpec((1, B, 2), mmap(0)),
            pl.BlockSpec((1, B, 2), mmap(1)),
            pl.BlockSpec((1, B, 2), mmap(2)),
            pl.BlockSpec((1, B, 2), mmap(3)),
            full((B, NLB)),
            full((B, AD)),
            full((1, NF)),
            full((1, NF)),
            pl.BlockSpec((512, HID), lambda k: (k, 0)),
            full((1, 128)),
            full((1, 128)),
            full((1, HID)),
            full((1, HID)),
            full((1, HID)),
            full((HID, AD)),
            full((1, AD)),
        ],
        out_specs=pl.BlockSpec((B, AD), lambda k: (0, 0)),
        out_shape=jax.ShapeDtypeStruct((B, AD), jnp.float32),
        scratch_shapes=[pltpu.VMEM((B, HID), jnp.float32)],
    )(f2d, f2d, f2d, f2d, m_r, m_r, m_r, m_r, flb, act, alb, blb, W1, at, bt,
      b1r, lnw, lnb, W4, b4r)


def kernel(feature_lb, feature_as, action, active_as, bn_as_w, bn_as_b,
           bn_lb_w, bn_lb_b, W1, b1, ln1_w, ln1_b, W4, b4):
    c, m_r = _sc_counts(active_as)

    f_r = feature_as.reshape(R, NF)
    c3 = c.reshape(NRT, 1, RT)
    st = _stats_call(c3, f_r, feature_lb)

    n_as = jnp.float32(B + B * NACT)
    mean_as = (st[0] + st[2]) / n_as
    var_as = (st[1] + st[3]) / n_as - mean_as * mean_as
    alpha_as = bn_as_w * lax.rsqrt(var_as + 1e-5)
    beta_as = bn_as_b - mean_as * alpha_as

    mean_lb = st[4] / B
    var_lb = st[5] / B - mean_lb * mean_lb
    alpha_lb = bn_lb_w * lax.rsqrt(var_lb + 1e-5)
    beta_lb = bn_lb_b - mean_lb * alpha_lb

    at = jnp.tile(alpha_as, 2)[None, :]
    bt = jnp.tile(beta_as, 2)[None, :]

    f2d = feature_as.reshape(B, AD * NF)
    return _mm_call(f2d, m_r, feature_lb, action, alpha_lb[None, :],
                    beta_lb[None, :], W1, at, bt, b1[None, :],
                    ln1_w[None, :], ln1_b[None, :], W4, b4[None, :])
